# Initial kernel scaffold; baseline (speedup 1.0000x reference)
import jax
import jax.numpy as jnp
from jax.experimental import pallas as pl


def kernel(inputs, target_table, context_table, W, b):
    raise NotImplementedError("write your pallas kernel here")


# speedup vs baseline: 3.0273x; 3.0273x over previous
import jax
import jax.numpy as jnp
from jax import lax
from jax.experimental import pallas as pl
from jax.experimental.pallas import tpu as pltpu
from jax.experimental.pallas import tpu_sc as plsc

EMB = 128
BATCH = 16384
LANES = 16
NC = 2
NS = 16
NW = NC * NS
BPW = BATCH // NW
CHUNK = 128
NCHUNK = BPW // CHUNK
NGROUP = BPW // LANES
WPAD = 2 * EMB + LANES


def _sc_body(inputs_hbm, ttab_hbm, ctab_hbm, wb_hbm, out_hbm,
             in_v, t_idx, c_idx, w_v, rows0, rows1, out_v, sem0, sem1):
    wid = lax.axis_index("s") * NC + lax.axis_index("c")
    base = wid * BPW

    pltpu.sync_copy(inputs_hbm.at[pl.ds(2 * base, 2 * BPW)], in_v)
    pltpu.sync_copy(wb_hbm, w_v)

    iota = lax.iota(jnp.int32, LANES)
    for g in range(NGROUP):
        pairs = 2 * (g * LANES + iota)
        t_idx[pl.ds(g * LANES, LANES)] = plsc.load_gather(in_v, [pairs])
        c_idx[pl.ds(g * LANES, LANES)] = plsc.load_gather(in_v, [pairs + 1])

    wt = [w_v[pl.ds(16 * j, 16)] for j in range(8)]
    wc = [w_v[pl.ds(EMB + 16 * j, 16)] for j in range(8)]
    bias = w_v[pl.ds(2 * EMB, LANES)][0]

    bufs = (rows0, rows1)
    sems = (sem0, sem1)

    def start(g):
        tab = ttab_hbm if g < NCHUNK else ctab_hbm
        idx = t_idx if g < NCHUNK else c_idx
        off = (g % NCHUNK) * CHUNK
        return pltpu.async_copy(tab.at[idx.at[pl.ds(off, CHUNK)]],
                                bufs[g % 2], sems[g % 2])

    handles = {0: start(0)}
    for g in range(2 * NCHUNK):
        if g + 1 < 2 * NCHUNK:
            handles[g + 1] = start(g + 1)
        handles[g].wait()
        buf = bufs[g % 2]
        w8 = wt if g < NCHUNK else wc
        base_off = (g % NCHUNK) * CHUNK
        is_ctx = g >= NCHUNK

        def grp_body(grp, carry):
            res = jnp.zeros((LANES,), jnp.float32)
            for rr in range(LANES):
                r = grp * LANES + rr
                acc = buf[r, pl.ds(0, 16)] * w8[0]
                for j in range(1, 8):
                    acc = acc + buf[r, pl.ds(16 * j, 16)] * w8[j]
                s = jnp.sum(acc)
                res = jnp.where(iota == rr, s, res)
            o = base_off + grp * LANES
            if is_ctx:
                out_v[pl.ds(o, LANES)] = out_v[pl.ds(o, LANES)] + res
            else:
                out_v[pl.ds(o, LANES)] = res
            return carry

        lax.fori_loop(0, CHUNK // LANES, grp_body, 0)

    for g in range(NGROUP):
        x = out_v[pl.ds(g * LANES, LANES)] + bias
        out_v[pl.ds(g * LANES, LANES)] = 1.0 / (1.0 + jnp.exp(-x))

    pltpu.sync_copy(out_v, out_hbm.at[pl.ds(base, BPW)])


def kernel(inputs, target_table, context_table, W, b):
    wb = jnp.concatenate([
        W.reshape(-1).astype(jnp.float32),
        b.reshape(-1).astype(jnp.float32),
        jnp.zeros((WPAD - 2 * EMB - 1,), jnp.float32),
    ])
    mesh = plsc.VectorSubcoreMesh(core_axis_name="c", subcore_axis_name="s")
    f = pl.kernel(
        _sc_body,
        mesh=mesh,
        compiler_params=pltpu.CompilerParams(needs_layout_passes=False),
        out_type=jax.ShapeDtypeStruct((BATCH,), jnp.float32),
        scratch_types=[
            pltpu.VMEM((2 * BPW,), jnp.int32),
            pltpu.VMEM((BPW,), jnp.int32),
            pltpu.VMEM((BPW,), jnp.int32),
            pltpu.VMEM((WPAD,), jnp.float32),
            pltpu.VMEM((CHUNK, EMB), jnp.float32),
            pltpu.VMEM((CHUNK, EMB), jnp.float32),
            pltpu.VMEM((BPW,), jnp.float32),
            pltpu.SemaphoreType.DMA,
            pltpu.SemaphoreType.DMA,
        ],
    )
    out = f(inputs.astype(jnp.int32).reshape(-1), target_table,
            context_table, wb)
    return out.reshape(BATCH, 1)
